# baseline (speedup 1.0000x reference)
import functools

import jax
import jax.numpy as jnp
from jax import lax
from jax.experimental import pallas as pl
from jax.experimental.pallas import tpu as pltpu
from jax.experimental.pallas import tpu_sc as plsc

_N = 10000
_E = 320000
_D = 128
_NC = 2
_NS = 16
_NW = _NC * _NS
_CHUNK = 16
_NCHUNKS = _E // _CHUNK
_NBUF = 4
_LEAD = 2
_CPT = 628
_NCHUNK_PAD = _NW * _CPT
_EPAD = _NCHUNK_PAD * _CHUNK
_ACC_ROWS = _N + 8
_ROWS_PER_TILE = 624
_TAIL_ROW0 = _NS * _ROWS_PER_TILE
_TAIL_ROWS = _N - _TAIL_ROW0
_BN_EPS = 1e-5


def _sc_segment_sum(x, edge_attr, src1d, dst1d, zeros):
    mesh = plsc.VectorSubcoreMesh(core_axis_name="c", subcore_axis_name="s")

    nbuf = _NBUF
    lead = _LEAD
    nsx = 4

    @functools.partial(
        pl.kernel,
        out_type=jax.ShapeDtypeStruct((_NC, _N, _D), jnp.float32),
        mesh=mesh,
        scratch_types=(
            [pltpu.VMEM((_CHUNK, _D), jnp.float32)] * nbuf +
            [pltpu.VMEM((_CHUNK, _D), jnp.float32)] * nbuf +
            [pltpu.VMEM((_CHUNK,), jnp.int32)] * nsx +
            [pltpu.VMEM((_CHUNK,), jnp.int32)] * nbuf +
            [pltpu.VMEM_SHARED((_ACC_ROWS, _D), jnp.float32)] +
            [pltpu.SemaphoreType.DMA] * (nsx + 4 * nbuf)
        ),
    )
    def k(x_hbm, ea_hbm, src_hbm, dst_hbm, z_hbm, out_hbm, *sc):
        it = iter(sc)

        def take(n):
            return [next(it) for _ in range(n)]

        xb = take(nbuf)
        eb = take(nbuf)
        sidx = take(nsx)
        didx = take(nbuf)
        acc = take(1)[0]
        qs = take(nsx)
        qd = take(nbuf)
        ix = take(nbuf)
        ie = take(nbuf)
        osem = take(nbuf)
        c = lax.axis_index("c")
        s = lax.axis_index("s")
        wid = c * _NS + s
        chunk0 = wid * _CPT

        def eoff(jj):
            return pl.multiple_of((chunk0 + jj) * _CHUNK, _CHUNK)

        def real(jj):
            return chunk0 + jj < _NCHUNKS

        def fire_sidx(jj, sx):
            pltpu.async_copy(src_hbm.at[pl.ds(eoff(jj), _CHUNK)], sidx[sx],
                             qs[sx])

        def wait_sidx(sx):
            pltpu.make_async_copy(src_hbm.at[pl.ds(0, _CHUNK)], sidx[sx],
                                  qs[sx]).wait()

        def fire_inputs(jj, sx, B):
            wait_sidx(sx)
            pltpu.async_copy(x_hbm.at[sidx[sx]], xb[B], ix[B])

            @pl.when(real(jj))
            def _():
                pltpu.async_copy(ea_hbm.at[pl.ds(eoff(jj), _CHUNK)], eb[B],
                                 ie[B])

            pltpu.async_copy(dst_hbm.at[pl.ds(eoff(jj), _CHUNK)], didx[B],
                             qd[B])

        def consume(jj, B):
            pltpu.make_async_copy(x_hbm.at[sidx[0]], xb[B], ix[B]).wait()

            @pl.when(real(jj))
            def _():
                pltpu.make_async_copy(ea_hbm.at[pl.ds(0, _CHUNK)], eb[B],
                                      ie[B]).wait()

            pltpu.make_async_copy(dst_hbm.at[pl.ds(0, _CHUNK)], didx[B],
                                  qd[B]).wait()
            for r in range(_CHUNK):
                for cc in range(_D // 16):
                    sl = (r, pl.ds(cc * 16, 16))
                    plsc.addupdate(xb[B].at[sl], eb[B][sl])
            pltpu.async_copy(xb[B], acc.at[didx[B]], osem[B], add=True)

        def drain(B):
            pltpu.make_async_copy(xb[B], acc.at[didx[B]], osem[B]).wait()

        for sx in range(nsx):
            fire_sidx(sx, sx)
        for jp in range(lead):
            fire_inputs(jp, jp % nsx, jp)

        row0 = s * _ROWS_PER_TILE
        pltpu.sync_copy(z_hbm.at[pl.ds(row0, _ROWS_PER_TILE)],
                        acc.at[pl.ds(row0, _ROWS_PER_TILE)])

        @pl.when(s == _NS - 1)
        def _():
            pltpu.sync_copy(z_hbm.at[pl.ds(_TAIL_ROW0, _TAIL_ROWS)],
                            acc.at[pl.ds(_TAIL_ROW0, _TAIL_ROWS)])

        plsc.subcore_barrier()

        def chunk_step(g, u):
            j = nbuf * g + u
            Bn = (u + lead) % nbuf

            @pl.when(j >= nbuf - lead)
            def _():
                drain(Bn)

            @pl.when(j + lead < _CPT)
            def _():
                fire_inputs(j + lead, (u + lead) % nsx, Bn)

            consume(j, u)

            @pl.when(j + nsx < _CPT)
            def _():
                fire_sidx(j + nsx, u % nsx)

        def group(g, carry):
            for u in range(nbuf):
                chunk_step(g, u)
            return carry

        lax.fori_loop(0, _CPT // nbuf, group, 0)

        for u in range(lead, nbuf):
            drain(u)

        plsc.subcore_barrier()
        pltpu.sync_copy(acc.at[pl.ds(row0, _ROWS_PER_TILE)],
                        out_hbm.at[c].at[pl.ds(row0, _ROWS_PER_TILE)])

        @pl.when(s == _NS - 1)
        def _():
            pltpu.sync_copy(acc.at[pl.ds(_TAIL_ROW0, _TAIL_ROWS)],
                            out_hbm.at[c].at[pl.ds(_TAIL_ROW0, _TAIL_ROWS)])

    return k(x, edge_attr, src1d, dst1d, zeros)


def _tc_mlp(parts, x, w1, g1, b1, w2, g2, b2, eps):
    def body(p_ref, x_ref, w1_ref, g1_ref, b1_ref, w2_ref, g2_ref, b2_ref,
             eps_ref, o_ref):
        scale = 1.0 + eps_ref[...]
        h = p_ref[0] + p_ref[1] + scale * x_ref[...]
        z = jnp.dot(h, w1_ref[...], preferred_element_type=jnp.float32)
        mu = jnp.mean(z, axis=0, keepdims=True)
        zc = z - mu
        var = jnp.mean(zc * zc, axis=0, keepdims=True)
        a = jnp.maximum(g1_ref[...] * zc * lax.rsqrt(var + _BN_EPS)
                        + b1_ref[...], 0.0)
        z2 = jnp.dot(a, w2_ref[...], preferred_element_type=jnp.float32)
        mu2 = jnp.mean(z2, axis=0, keepdims=True)
        zc2 = z2 - mu2
        var2 = jnp.mean(zc2 * zc2, axis=0, keepdims=True)
        o_ref[...] = jnp.maximum(g2_ref[...] * zc2 * lax.rsqrt(var2 + _BN_EPS)
                                 + b2_ref[...], 0.0)

    return pl.pallas_call(
        body,
        out_shape=jax.ShapeDtypeStruct((_N, _D), jnp.float32),
    )(parts, x, w1, g1, b1, w2, g2, b2, eps)


def kernel(node_rep, edge_attr, degree, W1, g1, b1, W2, g2, b2, epsilon,
           edge_index):
    del degree
    src = edge_index[0].astype(jnp.int32)
    dst = edge_index[1].astype(jnp.int32)
    npad = _EPAD - _E
    src1d = jnp.concatenate([src, jnp.zeros((npad,), jnp.int32)])
    dst1d = jnp.concatenate([dst, jnp.full((npad,), _N, jnp.int32)])
    zeros = jnp.zeros((_N, _D), jnp.float32)
    parts = _sc_segment_sum(node_rep, edge_attr, src1d, dst1d, zeros)
    return _tc_mlp(parts, node_rep,
                   W1, g1.reshape(1, -1), b1.reshape(1, -1),
                   W2, g2.reshape(1, -1), b2.reshape(1, -1),
                   epsilon.reshape(1, 1))


# speedup vs baseline: 1.5045x; 1.5045x over previous
import functools

import jax
import jax.numpy as jnp
from jax import lax
from jax.experimental import pallas as pl
from jax.experimental.pallas import tpu as pltpu
from jax.experimental.pallas import tpu_sc as plsc

_N = 10000
_E = 320000
_D = 128
_NC = 2
_NS = 16
_NW = _NC * _NS
_CHUNK = 16
_NCHUNKS = _E // _CHUNK
_NBUF = 8
_LEAD = 4
_CPT = 628
_NCHUNK_PAD = _NW * _CPT
_EPAD = _NCHUNK_PAD * _CHUNK
_ACC_ROWS = _N + 8
_ROWS_PER_TILE = 624
_TAIL_ROW0 = _NS * _ROWS_PER_TILE
_TAIL_ROWS = _N - _TAIL_ROW0
_BN_EPS = 1e-5


def _sc_segment_sum(x, edge_attr, src1d, dst1d, zeros):
    mesh = plsc.VectorSubcoreMesh(core_axis_name="c", subcore_axis_name="s")

    nbuf = _NBUF
    lead = _LEAD
    nsx = nbuf // 2
    jobs = 2 * _CPT

    @functools.partial(
        pl.kernel,
        out_type=jax.ShapeDtypeStruct((_NC, _N, _D), jnp.float32),
        mesh=mesh,
        scratch_types=(
            [pltpu.VMEM((_CHUNK, _D), jnp.float32)] * nbuf +
            [pltpu.VMEM((_CHUNK,), jnp.int32)] * nsx +
            [pltpu.VMEM((_CHUNK,), jnp.int32)] * nbuf +
            [pltpu.VMEM_SHARED((_ACC_ROWS, _D), jnp.float32)] +
            [pltpu.SemaphoreType.DMA] * (nsx + 3 * nbuf)
        ),
    )
    def k(x_hbm, ea_hbm, src_hbm, dst_hbm, z_hbm, out_hbm, *sc):
        it = iter(sc)

        def take(n):
            return [next(it) for _ in range(n)]

        buf = take(nbuf)
        sidx = take(nsx)
        didx = take(nbuf)
        acc = take(1)[0]
        qs = take(nsx)
        qd = take(nbuf)
        isem = take(nbuf)
        osem = take(nbuf)
        c = lax.axis_index("c")
        s = lax.axis_index("s")
        wid = c * _NS + s
        chunk0 = wid * _CPT

        def eoff(jj):
            return pl.multiple_of((chunk0 + jj) * _CHUNK, _CHUNK)

        def real(jj):
            return chunk0 + jj < _NCHUNKS

        def fire_sidx(jj, sx):
            pltpu.async_copy(src_hbm.at[pl.ds(eoff(jj), _CHUNK)], sidx[sx],
                             qs[sx])

        def wait_sidx(sx):
            pltpu.make_async_copy(src_hbm.at[pl.ds(0, _CHUNK)], sidx[sx],
                                  qs[sx]).wait()

        def fire_didx(jj, B):
            pltpu.async_copy(dst_hbm.at[pl.ds(eoff(jj), _CHUNK)], didx[B],
                             qd[B])

        def wait_didx(B):
            pltpu.make_async_copy(dst_hbm.at[pl.ds(0, _CHUNK)], didx[B],
                                  qd[B]).wait()

        def fire_gather(jj, sx, B):
            wait_sidx(sx)
            pltpu.async_copy(x_hbm.at[sidx[sx]], buf[B], isem[B])
            fire_didx(jj, B)

        def fire_eload(jj, B):
            @pl.when(real(jj))
            def _():
                pltpu.async_copy(ea_hbm.at[pl.ds(eoff(jj), _CHUNK)], buf[B],
                                 isem[B])
                fire_didx(jj, B)

        def scatter(B):
            pltpu.make_async_copy(ea_hbm.at[pl.ds(0, _CHUNK)], buf[B],
                                  isem[B]).wait()
            wait_didx(B)
            pltpu.async_copy(buf[B], acc.at[didx[B]], osem[B], add=True)

        def drain(B):
            pltpu.make_async_copy(buf[B], acc.at[didx[B]], osem[B]).wait()

        for sx in range(nsx):
            fire_sidx(sx, sx)
        for tp in range(lead):
            jj = tp // 2
            if tp % 2 == 0:
                fire_gather(jj, jj % nsx, tp)
            else:
                fire_eload(jj, tp)

        row0 = s * _ROWS_PER_TILE
        pltpu.sync_copy(z_hbm.at[pl.ds(row0, _ROWS_PER_TILE)],
                        acc.at[pl.ds(row0, _ROWS_PER_TILE)])

        @pl.when(s == _NS - 1)
        def _():
            pltpu.sync_copy(z_hbm.at[pl.ds(_TAIL_ROW0, _TAIL_ROWS)],
                            acc.at[pl.ds(_TAIL_ROW0, _TAIL_ROWS)])

        plsc.subcore_barrier()

        def job(g, u):
            t = nbuf * g + u
            jj = (nbuf // 2) * g + u // 2
            B = u
            Bn = (u + lead) % nbuf
            dj = lead // 2

            @pl.when(t >= nbuf - lead)
            def _():
                if u % 2 == 0:
                    drain(Bn)
                else:
                    @pl.when(real(jj - (nbuf - lead) // 2))
                    def _():
                        drain(Bn)

            @pl.when(t + lead < jobs)
            def _():
                if u % 2 == 0:
                    fire_gather(jj + dj, (u // 2 + dj) % nsx, Bn)
                else:
                    fire_eload(jj + dj, Bn)

            if u % 2 == 0:
                scatter(B)
                @pl.when(jj + nsx < _CPT)
                def _():
                    fire_sidx(jj + nsx, u // 2)
            else:
                @pl.when(real(jj))
                def _():
                    scatter(B)

        def group(g, carry):
            for u in range(nbuf):
                job(g, u)
            return carry

        lax.fori_loop(0, jobs // nbuf, group, 0)

        for u in range(lead, nbuf):
            jj = _CPT + (u - nbuf) // 2
            if u % 2 == 0:
                drain(u)
            else:
                @pl.when(real(jj))
                def _():
                    drain(u)

        plsc.subcore_barrier()
        pltpu.sync_copy(acc.at[pl.ds(row0, _ROWS_PER_TILE)],
                        out_hbm.at[c].at[pl.ds(row0, _ROWS_PER_TILE)])

        @pl.when(s == _NS - 1)
        def _():
            pltpu.sync_copy(acc.at[pl.ds(_TAIL_ROW0, _TAIL_ROWS)],
                            out_hbm.at[c].at[pl.ds(_TAIL_ROW0, _TAIL_ROWS)])

    return k(x, edge_attr, src1d, dst1d, zeros)


def _tc_mlp(parts, x, w1, g1, b1, w2, g2, b2, eps):
    def body(p_ref, x_ref, w1_ref, g1_ref, b1_ref, w2_ref, g2_ref, b2_ref,
             eps_ref, o_ref):
        scale = 1.0 + eps_ref[...]
        h = p_ref[0] + p_ref[1] + scale * x_ref[...]
        z = jnp.dot(h, w1_ref[...], preferred_element_type=jnp.float32)
        mu = jnp.mean(z, axis=0, keepdims=True)
        zc = z - mu
        var = jnp.mean(zc * zc, axis=0, keepdims=True)
        a = jnp.maximum(g1_ref[...] * zc * lax.rsqrt(var + _BN_EPS)
                        + b1_ref[...], 0.0)
        z2 = jnp.dot(a, w2_ref[...], preferred_element_type=jnp.float32)
        mu2 = jnp.mean(z2, axis=0, keepdims=True)
        zc2 = z2 - mu2
        var2 = jnp.mean(zc2 * zc2, axis=0, keepdims=True)
        o_ref[...] = jnp.maximum(g2_ref[...] * zc2 * lax.rsqrt(var2 + _BN_EPS)
                                 + b2_ref[...], 0.0)

    return pl.pallas_call(
        body,
        out_shape=jax.ShapeDtypeStruct((_N, _D), jnp.float32),
    )(parts, x, w1, g1, b1, w2, g2, b2, eps)


def kernel(node_rep, edge_attr, degree, W1, g1, b1, W2, g2, b2, epsilon,
           edge_index):
    del degree
    src = edge_index[0].astype(jnp.int32)
    dst = edge_index[1].astype(jnp.int32)
    npad = _EPAD - _E
    src1d = jnp.concatenate([src, jnp.zeros((npad,), jnp.int32)])
    dst1d = jnp.concatenate([dst, jnp.full((npad,), _N, jnp.int32)])
    zeros = jnp.zeros((_N, _D), jnp.float32)
    parts = _sc_segment_sum(node_rep, edge_attr, src1d, dst1d, zeros)
    return _tc_mlp(parts, node_rep,
                   W1, g1.reshape(1, -1), b1.reshape(1, -1),
                   W2, g2.reshape(1, -1), b2.reshape(1, -1),
                   epsilon.reshape(1, 1))
